# baseline (speedup 1.0000x reference)
import functools

import jax
import jax.numpy as jnp
from jax import lax
from jax.experimental import pallas as pl
from jax.experimental.pallas import tpu as pltpu
from jax.experimental.pallas import tpu_sc as plsc

_NUM_WORKERS = 32
_L = 200
_BB = 8
_D = 64
_NP = _L * _BB
_PPW = _NP // _NUM_WORKERS


def _make_gather(v):
  mesh = plsc.VectorSubcoreMesh(core_axis_name="c", subcore_axis_name="s")

  @functools.partial(
      pl.kernel,
      mesh=mesh,
      out_type=jax.ShapeDtypeStruct((_L, 8, _BB, 8, 128), jnp.float32),
      scratch_types=[
          pltpu.VMEM((_PPW * 128,), jnp.int32),
          pltpu.VMEM((128, _D), jnp.float32),
          pltpu.VMEM((128, _D), jnp.float32),
          pltpu.VMEM((8, 8, 128), jnp.float32),
          pltpu.VMEM((8, 8, 128), jnp.float32),
          pltpu.SemaphoreType.DMA,
          pltpu.SemaphoreType.DMA,
          pltpu.SemaphoreType.DMA,
          pltpu.SemaphoreType.DMA,
      ],
      compiler_params=pltpu.CompilerParams(
          use_tc_tiling_on_sc=False, needs_layout_passes=False),
  )
  def gather(table_hbm, idx_hbm, out_hbm, idx_v, rows0, rows1, t0, t1,
             gsem0, gsem1, ssem0, ssem1):
    wid = lax.axis_index("s") * 2 + lax.axis_index("c")
    p_base = wid * _PPW
    rows = (rows0, rows1)
    tbuf = (t0, t1)
    gsem = (gsem0, gsem1)
    ssem = (ssem0, ssem1)
    iota16 = lax.iota(jnp.int32, 16)
    rix = [iota16 + 16 * blk for blk in range(8)]

    pltpu.sync_copy(idx_hbm.at[pl.ds(p_base * 128, _PPW * 128)], idx_v)

    def issue_gather(k, cur):
      pltpu.async_copy(
          table_hbm.at[idx_v.at[pl.ds(k * 128, 128)]], rows[cur], gsem[cur])

    def emit(k, cur):
      p = p_base + k
      l = p // _BB
      bh = p % _BB
      pltpu.make_async_copy(
          table_hbm.at[idx_v.at[pl.ds(k * 128, 128)]], rows[cur],
          gsem[cur]).wait()
      @pl.when(k + 1 < _PPW)
      def _():
        issue_gather(k + 1, 1 - cur)
      @pl.when(k >= 2)
      def _():
        pltpu.make_async_copy(
            tbuf[cur], out_hbm.at[l, :, bh], ssem[cur]).wait()
      for dh in range(8):
        for dl in range(8):
          c = jnp.full((16,), dh * 8 + dl, jnp.int32)
          vecs = [
              plsc.load_gather(rows[cur], [rix[blk], c])
              for blk in range(8)
          ]
          for blk in range(8):
            tbuf[cur][dh, dl, pl.ds(blk * 16, 16)] = vecs[blk]
      pltpu.async_copy(tbuf[cur], out_hbm.at[l, :, bh], ssem[cur])

    issue_gather(0, 0)

    def super_body(i, carry):
      emit(2 * i, 0)
      emit(2 * i + 1, 1)
      return carry

    lax.fori_loop(0, _PPW // 2, super_body, 0)

    p_last = p_base + _PPW - 1
    for cur, p in ((0, p_last - 1), (1, p_last)):
      pltpu.make_async_copy(
          tbuf[cur], out_hbm.at[p // _BB, :, p % _BB], ssem[cur]).wait()

  return gather


def kernel(x, pe):
  b, l = x.shape
  v = pe.shape[0]
  d = pe.shape[-1]
  xt = x.T.reshape(b * l)
  table = pe.reshape(v, d)
  a = _make_gather(v)(table, xt)
  return a.transpose(2, 4, 0, 1, 3).reshape(b, l, 1, d)


# speedup vs baseline: 2.3844x; 2.3844x over previous
import functools

import jax
import jax.numpy as jnp
from jax import lax
from jax.experimental import pallas as pl
from jax.experimental.pallas import tpu as pltpu
from jax.experimental.pallas import tpu_sc as plsc

_NUM_WORKERS = 32
_L = 200
_BB = 8
_D = 64
_NP = _L * _BB
_PPW = _NP // _NUM_WORKERS
_PPC = 5
_NCH = _PPW // _PPC
_P = 129


def _make_gather(v):
  mesh = plsc.VectorSubcoreMesh(core_axis_name="c", subcore_axis_name="s")

  @functools.partial(
      pl.kernel,
      mesh=mesh,
      out_type=jax.ShapeDtypeStruct((_L, 8, _BB, 8, 128), jnp.float32),
      scratch_types=[
          pltpu.VMEM((_PPW * 128,), jnp.int32),
          pltpu.VMEM((2, _PPC * 128, _D), jnp.float32),
          pltpu.VMEM((2, 8, 8, _P), jnp.float32),
          pltpu.SemaphoreType.DMA((2,)),
          pltpu.SemaphoreType.DMA((2,)),
      ],
      compiler_params=pltpu.CompilerParams(
          use_tc_tiling_on_sc=False, needs_layout_passes=False),
  )
  def gather(table_hbm, idx_hbm, out_hbm, idx_v, rows, tbuf, gsem, ssem):
    wid = lax.axis_index("s") * 2 + lax.axis_index("c")
    p_base = wid * _PPW
    iota16 = lax.iota(jnp.int32, 16)
    dh0 = lax.shift_right_logical(iota16, 1 + 2)
    dlv = lax.bitwise_and(iota16, jnp.full((16,), 7, jnp.int32))
    dhv = [dh0 + 2 * k for k in range(4)]

    pltpu.sync_copy(idx_hbm.at[pl.ds(p_base * 128, _PPW * 128)], idx_v)

    def issue_gather(c):
      par = lax.rem(c, 2)
      pltpu.async_copy(
          table_hbm.at[idx_v.at[pl.ds(c * (_PPC * 128), _PPC * 128)]],
          rows.at[par], gsem.at[par])

    issue_gather(0)

    def body(k, carry):
      c = k // _PPC
      j = lax.rem(k, _PPC)
      par = lax.rem(c, 2)
      tpar = lax.rem(k, 2)
      p = p_base + k
      l = p // _BB
      bh = lax.rem(p, _BB)

      @pl.when(j == 0)
      def _():
        pltpu.make_async_copy(
            table_hbm.at[idx_v.at[pl.ds(c * (_PPC * 128), _PPC * 128)]],
            rows.at[par], gsem.at[par]).wait()

        @pl.when(c + 1 < _NCH)
        def _():
          issue_gather(c + 1)

      @pl.when(k >= 2)
      def _():
        pltpu.make_async_copy(
            tbuf.at[tpar, :, :, pl.ds(0, 128)], out_hbm.at[l, :, bh],
            ssem.at[tpar]).wait()

      row0 = j * 128
      for b in range(128):
        bv = jnp.full((16,), b, jnp.int32)
        vecs = [rows[par, row0 + b, pl.ds(16 * k2, 16)] for k2 in range(4)]
        for k2 in range(4):
          plsc.store_scatter(tbuf.at[tpar], [dhv[k2], dlv, bv], vecs[k2])

      pltpu.async_copy(
          tbuf.at[tpar, :, :, pl.ds(0, 128)], out_hbm.at[l, :, bh],
          ssem.at[tpar])
      return carry

    lax.fori_loop(0, _PPW, body, 0)

    for k in (_PPW - 2, _PPW - 1):
      p = p_base + k
      pltpu.make_async_copy(
          tbuf.at[lax.rem(k, 2), :, :, pl.ds(0, 128)],
          out_hbm.at[p // _BB, :, lax.rem(p, _BB)],
          ssem.at[lax.rem(k, 2)]).wait()

  return gather


def kernel(x, pe):
  b, l = x.shape
  v = pe.shape[0]
  d = pe.shape[-1]
  xt = x.T.reshape(b * l)
  table = pe.reshape(v, d)
  a = _make_gather(v)(table, xt)
  return a.transpose(2, 4, 0, 1, 3).reshape(b, l, 1, d)
